# baseline (speedup 1.0000x reference)
import jax
import jax.numpy as jnp
from jax.experimental import pallas as pl

_EGO = 0.01831563888873418
_EGE = 0.36787944117144233
_NEG = -1e30
_B = 16
_L = 256
_CHUNK = 32
_NCHUNK = 16


def _shiftn_by(x, k):
    return jnp.concatenate(
        [jnp.full((x.shape[0], k), _NEG, x.dtype), x[:, :-k]], axis=1)


def _shift0(x):
    return jnp.concatenate(
        [jnp.zeros((x.shape[0], 1), x.dtype), x[:, :-1]], axis=1)


def _rotk(x, k):
    k = k % _L
    if k == 0:
        return x
    return jnp.concatenate([x[:, -k:], x[:, :-k]], axis=1)


def _sel4(t, v):
    return jnp.where(t == 0, v[0],
           jnp.where(t == 1, v[1],
           jnp.where(t == 2, v[2], v[3])))


def _sw_kernel(predT_ref, v0_ref, out_ref):
    predT = predT_ref[...]
    v = v0_ref[...]
    lane = jax.lax.broadcasted_iota(jnp.int32, (_B, _L), 1)
    vmask = lane < (_L - 1)
    zero = jnp.zeros((_B, _L), jnp.float32)

    p0p = [jnp.where(vmask, jnp.maximum(predT[p], 0.0), zero) for p in range(4)]
    ep1 = [jnp.where(vmask,
                     jnp.exp(jnp.concatenate(
                         [predT[p][:, 1:], predT[p][:, :1]], axis=1)),
                     zero) for p in range(4)]

    def chunk(i, carry):
        (vc, mx, ea1, er1, ssq1, seg1, seg2, eg1, w1, acc) = carry
        mxe = mx + jnp.log(jnp.maximum(eg1, 1.0))
        mxn = mxe
        for j in (1, 2, 4, 8, 16, 32):
            mxn = jnp.maximum(mxn, _shiftn_by(mxn, j))
        rb = jnp.exp(mx - mxn)
        dsh = jnp.exp(jnp.minimum(_shiftn_by(mxn, 1) - mxn, 80.0))
        e00 = jnp.exp(-mxn)
        ea1 = ea1 * rb
        er1 = er1 * rb
        ssq1 = ssq1 * rb
        seg1 = seg1 * rb
        seg2 = seg2 * rb
        w1 = w1 * rb
        acc = acc * rb

        d_base = i * _CHUNK
        ld0 = lane - d_base
        ws = [_rotk(vc, k) for k in range(_CHUNK + 1)]
        for k in range(_CHUNK):
            ld = ld0 - k
            mask = (ld <= 0) & (ld >= -254)
            sp = jnp.where(mask, _sel4(ws[k], p0p), zero)
            esmx = jnp.where(mask, jnp.exp(sp), zero)
            exe = jnp.where(mask, _sel4(ws[k + 1], ep1), zero)
            ea0 = esmx * (seg2 + e00)
            er0 = w1 + _EGE * er1
            ed0 = ssq1
            eg0 = ea0 + er0 + ed0
            w0 = _EGO * ea0
            sq0 = w0 + _EGO * er0 + _EGE * ed0
            acc = acc + eg0 * exe
            seg2 = seg1
            seg1 = _shift0(eg0) * dsh
            ssq1 = _shift0(sq0) * dsh
            ea1, er1, w1, eg1 = ea0, er0, w0, eg0
        vc = ws[_CHUNK]
        return (vc, mxn, ea1, er1, ssq1, seg1, seg2, eg1, w1, acc)

    zi = predT[0] * 0.0
    init = (v, zi, zi, zi, zi, zi, zi, zi, zi, zi)
    out = jax.lax.fori_loop(0, _NCHUNK, chunk, init)
    mx, acc = out[1], out[9]

    t = mx + jnp.log(jnp.maximum(acc, 1e-35))
    mb = jnp.max(t, axis=1, keepdims=True)
    sb = jnp.sum(jnp.exp(t - mb), axis=1, keepdims=True)
    fin = mb + jnp.log(sb)
    out_ref[...] = jnp.full((1, 1), -jnp.sum(fin) * (1.0 / _B), jnp.float32)


def _prep(predictions, targets):
    predT = jnp.transpose(predictions.astype(jnp.float32), (2, 0, 1))
    t = targets.astype(jnp.int32)
    v0 = jnp.concatenate([t[:, :1], jnp.flip(t[:, 1:], axis=1)], axis=1)
    return predT, v0


@jax.jit
def kernel(predictions, targets):
    predT, v0 = _prep(predictions, targets)
    out = pl.pallas_call(
        _sw_kernel,
        out_shape=jax.ShapeDtypeStruct((1, 1), jnp.float32),
    )(predT, v0)
    return out[0, 0]
